# baseline (speedup 1.0000x reference)
import jax
import jax.numpy as jnp
from jax import lax
from jax.experimental import pallas as pl
from jax.experimental.pallas import tpu as pltpu
from jax.experimental.pallas import tpu_sc as plsc

D = 768
M = 64
B = 16
K = 2
EPS = 1e-8
LAMBDA = 1.0

TN = 1024
MP = 128


def _stage_a(xf_ref, v_ref, e_ref, vnb_ref):
    pid = pl.program_id(0)

    @pl.when(pid == 0)
    def _():
        v2 = v_ref[...]
        rv = 1.0 / jnp.maximum(
            jnp.sqrt(jnp.sum(v2 * v2, axis=0, keepdims=True)), EPS)
        vnb_ref[...] = (v2 * rv).astype(jnp.bfloat16)

    xt = xf_ref[...]
    xn = xt / jnp.maximum(jnp.sqrt(jnp.sum(xt * xt, axis=1, keepdims=True)), EPS)
    h = jnp.dot(xn.astype(jnp.bfloat16), vnb_ref[...],
                preferred_element_type=jnp.float32)
    jj = jax.lax.broadcasted_iota(jnp.int32, (M * B, M), 0) // B
    mcol = jax.lax.broadcasted_iota(jnp.int32, (M * B, M), 1)
    S = (jj == mcol).astype(jnp.bfloat16)
    h2 = h * h
    h2a = h2.astype(jnp.bfloat16)
    r1 = h2 - h2a.astype(jnp.float32)
    h2b = r1.astype(jnp.bfloat16)
    h2c = (r1 - h2b.astype(jnp.float32)).astype(jnp.bfloat16)
    dn = (((0,), (1,)), ((), ()))
    e_ref[...] = (
        jax.lax.dot_general(S, h2a, dn, preferred_element_type=jnp.float32)
        + jax.lax.dot_general(S, h2b, dn, preferred_element_type=jnp.float32)
        + jax.lax.dot_general(S, h2c, dn, preferred_element_type=jnp.float32))


def _sc_top2(energy_hbm, mask_hbm, e_v, m_v):
    info = plsc.get_sparse_core_info()
    nc = info.num_cores
    wid = lax.axis_index("s") * nc + lax.axis_index("c")
    rows = 128
    nw = 2048 // rows
    base = wid * rows

    @pl.when(wid < nw)
    def _():
        pltpu.sync_copy(energy_hbm.at[:, pl.ds(base, rows)], e_v)
        iota = jax.lax.broadcasted_iota(jnp.int32, (16,), 0)
        for c in range(rows // 16):
            def mbody(m, carry):
                vmax, idx1, vmax2, idx2 = carry
                em = e_v[m, pl.ds(16 * c, 16)]
                t2v = jnp.where(em > vmax2, em, vmax2)
                t2i = jnp.where(em > vmax2, m, idx2)
                nvmax2 = jnp.where(em > vmax, vmax, t2v)
                nidx2 = jnp.where(em > vmax, idx1, t2i)
                nvmax = jnp.where(em > vmax, em, vmax)
                nidx1 = jnp.where(em > vmax, m, idx1)
                return nvmax, nidx1, nvmax2, nidx2
            neg = jnp.full((16,), -1.0, dtype=jnp.float32)
            zero = iota * 0
            vmax, idx1, vmax2, idx2 = lax.fori_loop(0, M, mbody, (neg, zero, neg, zero))

            def wbody(m, carry):
                i1, i2 = carry
                mk = jnp.where(i1 == m, 1.0, jnp.where(i2 == m, 1.0, 0.0))
                m_v[m, pl.ds(16 * c, 16)] = mk
                return i1, i2

            lax.fori_loop(0, M, wbody, (idx1, idx2))
        pltpu.sync_copy(m_v, mask_hbm.at[:, pl.ds(base, rows)])


def _stage_b(xf_ref, v_ref, ud_ref, ulast_ref, e_ref, mask_ref,
             xout_ref, stats_ref, vnb_ref, unb_ref, eacc_ref, cacc_ref,
             sacc_ref):
    pid = pl.program_id(0)
    nprog = pl.num_programs(0)
    n_tok = nprog * TN

    @pl.when(pid == 0)
    def _():
        v2 = v_ref[...]
        ud = ud_ref[...]
        ulast = ulast_ref[...]
        rv = 1.0 / jnp.maximum(
            jnp.sqrt(jnp.sum(v2 * v2, axis=0, keepdims=True)), EPS)
        ru = 1.0 / jnp.maximum(
            jnp.sqrt(jnp.sum(ud * ud, axis=0, keepdims=True) + ulast * ulast),
            EPS)
        vnb_ref[...] = (v2 * rv).astype(jnp.bfloat16)
        unb_ref[...] = (ud * ru).astype(jnp.bfloat16)
        eacc_ref[...] = jnp.zeros_like(eacc_ref)
        cacc_ref[...] = jnp.zeros_like(cacc_ref)
        for i in range(4):
            sacc_ref[i] = 0.0

    vnb = vnb_ref[...]
    unb = unb_ref[...]
    energyT = e_ref[...]
    maskT = mask_ref[...]

    xt = xf_ref[...]
    xn = xt / jnp.maximum(jnp.sqrt(jnp.sum(xt * xt, axis=1, keepdims=True)), EPS)
    h = jnp.dot(xn.astype(jnp.bfloat16), vnb,
                preferred_element_type=jnp.float32)

    jj2 = jax.lax.broadcasted_iota(jnp.int32, (M, M * B), 1) // B
    mrow = jax.lax.broadcasted_iota(jnp.int32, (M, M * B), 0)
    S2 = (jj2 == mrow).astype(jnp.bfloat16)
    mask_b = jax.lax.dot_general(maskT.astype(jnp.bfloat16), S2,
                                 (((0,), (0,)), ((), ())),
                                 preferred_element_type=jnp.float32)
    hmb = h.astype(jnp.bfloat16) * mask_b.astype(jnp.bfloat16)

    x_hat = jax.lax.dot_general(hmb, vnb, (((1,), (1,)), ((), ())),
                                preferred_element_type=jnp.float32)
    writes = jax.lax.dot_general(hmb, unb, (((1,), (1,)), ((), ())),
                                 preferred_element_type=jnp.float32)
    g = jnp.dot(writes.astype(jnp.bfloat16), unb,
                preferred_element_type=jnp.float32)

    resid = xn - x_hat
    xo = xn + LAMBDA * writes
    xout_ref[...] = xo / jnp.maximum(
        jnp.sqrt(jnp.sum(xo * xo, axis=1, keepdims=True)), EPS)

    eacc_ref[...] += jnp.sum(energyT, axis=1, keepdims=True)
    cacc_ref[...] += jnp.sum(maskT, axis=1, keepdims=True)
    diff = g - h
    sacc_ref[0] += jnp.sum(resid * resid)
    sacc_ref[1] += jnp.sum(x_hat * x_hat)
    sacc_ref[2] += jnp.sum(maskT * energyT)
    sacc_ref[3] += jnp.sum(mask_b * diff * diff)

    @pl.when(pid == nprog - 1)
    def _():
        nf = jnp.float32(n_tok)
        uncaptured = sacc_ref[0] / nf
        recon = sacc_ref[1] / nf
        captured = sacc_ref[2] / nf
        writer = sacc_ref[3] / (nf * jnp.float32(K * B))
        avg_e = eacc_ref[...] / nf
        denom = jnp.maximum(jnp.sum(avg_e), EPS)
        probs = jnp.maximum(avg_e / denom, EPS)
        entropy = -jnp.sum(probs * jnp.log(probs)) / jnp.log(jnp.float32(M))
        counts = cacc_ref[...]
        expected = jnp.float32(K) / jnp.float32(M) * nf
        n_low = jnp.sum(jnp.where(counts <= 0.1 * expected, 1.0, 0.0))
        n_dead = jnp.sum(jnp.where(counts <= 0.01 * expected, 1.0, 0.0))
        stats_ref[0] = uncaptured + writer
        stats_ref[1] = uncaptured
        stats_ref[2] = entropy
        stats_ref[3] = captured
        stats_ref[4] = recon
        stats_ref[5] = n_low
        stats_ref[6] = n_dead


def kernel(x, V, U):
    n_tok = x.shape[0] * x.shape[1]
    grid = n_tok // TN
    xf = x.reshape(n_tok, D)
    v2 = V.reshape(D, M * B)
    u_t = jnp.transpose(U, (1, 0, 2)).reshape(D + 1, M * B)
    ud = u_t[:D]
    ulast = u_t[D:]

    energyT = pl.pallas_call(
        _stage_a,
        grid=(grid,),
        in_specs=[
            pl.BlockSpec((TN, D), lambda i: (i, 0)),
            pl.BlockSpec((D, M * B), lambda i: (0, 0)),
        ],
        out_specs=pl.BlockSpec((M, TN), lambda i: (0, i)),
        out_shape=jax.ShapeDtypeStruct((M, n_tok), jnp.float32),
        scratch_shapes=[pltpu.VMEM((D, M * B), jnp.bfloat16)],
    )(xf, v2)

    mesh = plsc.VectorSubcoreMesh(core_axis_name="c", subcore_axis_name="s")
    maskT = pl.kernel(
        _sc_top2,
        mesh=mesh,
        out_type=jax.ShapeDtypeStruct((M, n_tok), jnp.float32),
        scratch_types=[
            pltpu.VMEM((M, 128), jnp.float32),
            pltpu.VMEM((M, 128), jnp.float32),
        ],
    )(energyT)

    x_out, stats = pl.pallas_call(
        _stage_b,
        grid=(grid,),
        in_specs=[
            pl.BlockSpec((TN, D), lambda i: (i, 0)),
            pl.BlockSpec((D, M * B), lambda i: (0, 0)),
            pl.BlockSpec((D, M * B), lambda i: (0, 0)),
            pl.BlockSpec((1, M * B), lambda i: (0, 0)),
            pl.BlockSpec((M, TN), lambda i: (0, i)),
            pl.BlockSpec((M, TN), lambda i: (0, i)),
        ],
        out_specs=[
            pl.BlockSpec((TN, D), lambda i: (i, 0)),
            pl.BlockSpec(memory_space=pltpu.SMEM),
        ],
        out_shape=[
            jax.ShapeDtypeStruct((n_tok, D), jnp.float32),
            jax.ShapeDtypeStruct((8,), jnp.float32),
        ],
        scratch_shapes=[
            pltpu.VMEM((D, M * B), jnp.bfloat16),
            pltpu.VMEM((D, M * B), jnp.bfloat16),
            pltpu.VMEM((M, 1), jnp.float32),
            pltpu.VMEM((M, 1), jnp.float32),
            pltpu.SMEM((8,), jnp.float32),
        ],
    )(xf, v2, ud, ulast, energyT, maskT)

    x_out = x_out.reshape(x.shape)
    return (x_out, stats[0], stats[1], stats[2], stats[3], stats[4],
            stats[5], stats[6])


# speedup vs baseline: 1.0106x; 1.0106x over previous
import jax
import jax.numpy as jnp
from jax import lax
from jax.experimental import pallas as pl
from jax.experimental.pallas import tpu as pltpu
from jax.experimental.pallas import tpu_sc as plsc

D = 768
M = 64
B = 16
K = 2
EPS = 1e-8
LAMBDA = 1.0

TN = 512
MP = 128


def _stage_a(xf_ref, v_ref, e_ref, vnb_ref):
    pid = pl.program_id(0)

    @pl.when(pid == 0)
    def _():
        v2 = v_ref[...]
        rv = 1.0 / jnp.maximum(
            jnp.sqrt(jnp.sum(v2 * v2, axis=0, keepdims=True)), EPS)
        vnb_ref[...] = (v2 * rv).astype(jnp.bfloat16)

    xt = xf_ref[...]
    xn = xt / jnp.maximum(jnp.sqrt(jnp.sum(xt * xt, axis=1, keepdims=True)), EPS)
    h = jnp.dot(xn.astype(jnp.bfloat16), vnb_ref[...],
                preferred_element_type=jnp.float32)
    jj = jax.lax.broadcasted_iota(jnp.int32, (M * B, M), 0) // B
    mcol = jax.lax.broadcasted_iota(jnp.int32, (M * B, M), 1)
    S = (jj == mcol).astype(jnp.bfloat16)
    h2 = h * h
    h2a = h2.astype(jnp.bfloat16)
    r1 = h2 - h2a.astype(jnp.float32)
    h2b = r1.astype(jnp.bfloat16)
    h2c = (r1 - h2b.astype(jnp.float32)).astype(jnp.bfloat16)
    dn = (((0,), (1,)), ((), ()))
    e_ref[...] = (
        jax.lax.dot_general(S, h2a, dn, preferred_element_type=jnp.float32)
        + jax.lax.dot_general(S, h2b, dn, preferred_element_type=jnp.float32)
        + jax.lax.dot_general(S, h2c, dn, preferred_element_type=jnp.float32))


def _sc_top2(energy_hbm, mask_hbm, e_v, m_v):
    info = plsc.get_sparse_core_info()
    nc = info.num_cores
    wid = lax.axis_index("s") * nc + lax.axis_index("c")
    rows = 128
    nw = 2048 // rows
    base = wid * rows

    @pl.when(wid < nw)
    def _():
        pltpu.sync_copy(energy_hbm.at[:, pl.ds(base, rows)], e_v)
        iota = jax.lax.broadcasted_iota(jnp.int32, (16,), 0)
        for c in range(rows // 16):
            def mbody(m, carry):
                vmax, idx1, vmax2, idx2 = carry
                em = e_v[m, pl.ds(16 * c, 16)]
                t2v = jnp.where(em > vmax2, em, vmax2)
                t2i = jnp.where(em > vmax2, m, idx2)
                nvmax2 = jnp.where(em > vmax, vmax, t2v)
                nidx2 = jnp.where(em > vmax, idx1, t2i)
                nvmax = jnp.where(em > vmax, em, vmax)
                nidx1 = jnp.where(em > vmax, m, idx1)
                return nvmax, nidx1, nvmax2, nidx2
            neg = jnp.full((16,), -1.0, dtype=jnp.float32)
            zero = iota * 0
            vmax, idx1, vmax2, idx2 = lax.fori_loop(0, M, mbody, (neg, zero, neg, zero))

            def wbody(m, carry):
                i1, i2 = carry
                mk = jnp.where(i1 == m, 1.0, jnp.where(i2 == m, 1.0, 0.0))
                m_v[m, pl.ds(16 * c, 16)] = mk
                return i1, i2

            lax.fori_loop(0, M, wbody, (idx1, idx2))
        pltpu.sync_copy(m_v, mask_hbm.at[:, pl.ds(base, rows)])


def _stage_b(xf_ref, v_ref, ud_ref, ulast_ref, e_ref, mask_ref,
             xout_ref, stats_ref, vnb_ref, unb_ref, eacc_ref, cacc_ref,
             sacc_ref):
    pid = pl.program_id(0)
    nprog = pl.num_programs(0)
    n_tok = nprog * TN

    @pl.when(pid == 0)
    def _():
        v2 = v_ref[...]
        ud = ud_ref[...]
        ulast = ulast_ref[...]
        rv = 1.0 / jnp.maximum(
            jnp.sqrt(jnp.sum(v2 * v2, axis=0, keepdims=True)), EPS)
        ru = 1.0 / jnp.maximum(
            jnp.sqrt(jnp.sum(ud * ud, axis=0, keepdims=True) + ulast * ulast),
            EPS)
        vnb_ref[...] = (v2 * rv).astype(jnp.bfloat16)
        unb_ref[...] = (ud * ru).astype(jnp.bfloat16)
        eacc_ref[...] = jnp.zeros_like(eacc_ref)
        cacc_ref[...] = jnp.zeros_like(cacc_ref)
        for i in range(4):
            sacc_ref[i] = 0.0

    vnb = vnb_ref[...]
    unb = unb_ref[...]
    energyT = e_ref[...]
    maskT = mask_ref[...]

    xt = xf_ref[...]
    xn = xt / jnp.maximum(jnp.sqrt(jnp.sum(xt * xt, axis=1, keepdims=True)), EPS)
    h = jnp.dot(xn.astype(jnp.bfloat16), vnb,
                preferred_element_type=jnp.float32)

    jj2 = jax.lax.broadcasted_iota(jnp.int32, (M, M * B), 1) // B
    mrow = jax.lax.broadcasted_iota(jnp.int32, (M, M * B), 0)
    S2 = (jj2 == mrow).astype(jnp.bfloat16)
    mask_b = jax.lax.dot_general(maskT.astype(jnp.bfloat16), S2,
                                 (((0,), (0,)), ((), ())),
                                 preferred_element_type=jnp.float32)
    hmb = h.astype(jnp.bfloat16) * mask_b.astype(jnp.bfloat16)

    x_hat = jax.lax.dot_general(hmb, vnb, (((1,), (1,)), ((), ())),
                                preferred_element_type=jnp.float32)
    writes = jax.lax.dot_general(hmb, unb, (((1,), (1,)), ((), ())),
                                 preferred_element_type=jnp.float32)
    g = jnp.dot(writes.astype(jnp.bfloat16), unb,
                preferred_element_type=jnp.float32)

    resid = xn - x_hat
    xo = xn + LAMBDA * writes
    xout_ref[...] = xo / jnp.maximum(
        jnp.sqrt(jnp.sum(xo * xo, axis=1, keepdims=True)), EPS)

    eacc_ref[...] += jnp.sum(energyT, axis=1, keepdims=True)
    cacc_ref[...] += jnp.sum(maskT, axis=1, keepdims=True)
    diff = g - h
    sacc_ref[0] += jnp.sum(resid * resid)
    sacc_ref[1] += jnp.sum(x_hat * x_hat)
    sacc_ref[2] += jnp.sum(maskT * energyT)
    sacc_ref[3] += jnp.sum(mask_b * diff * diff)

    @pl.when(pid == nprog - 1)
    def _():
        nf = jnp.float32(n_tok)
        uncaptured = sacc_ref[0] / nf
        recon = sacc_ref[1] / nf
        captured = sacc_ref[2] / nf
        writer = sacc_ref[3] / (nf * jnp.float32(K * B))
        avg_e = eacc_ref[...] / nf
        denom = jnp.maximum(jnp.sum(avg_e), EPS)
        probs = jnp.maximum(avg_e / denom, EPS)
        entropy = -jnp.sum(probs * jnp.log(probs)) / jnp.log(jnp.float32(M))
        counts = cacc_ref[...]
        expected = jnp.float32(K) / jnp.float32(M) * nf
        n_low = jnp.sum(jnp.where(counts <= 0.1 * expected, 1.0, 0.0))
        n_dead = jnp.sum(jnp.where(counts <= 0.01 * expected, 1.0, 0.0))
        stats_ref[0] = uncaptured + writer
        stats_ref[1] = uncaptured
        stats_ref[2] = entropy
        stats_ref[3] = captured
        stats_ref[4] = recon
        stats_ref[5] = n_low
        stats_ref[6] = n_dead


def kernel(x, V, U):
    n_tok = x.shape[0] * x.shape[1]
    grid = n_tok // TN
    xf = x.reshape(n_tok, D)
    v2 = V.reshape(D, M * B)
    u_t = jnp.transpose(U, (1, 0, 2)).reshape(D + 1, M * B)
    ud = u_t[:D]
    ulast = u_t[D:]

    energyT = pl.pallas_call(
        _stage_a,
        grid=(grid,),
        in_specs=[
            pl.BlockSpec((TN, D), lambda i: (i, 0)),
            pl.BlockSpec((D, M * B), lambda i: (0, 0)),
        ],
        out_specs=pl.BlockSpec((M, TN), lambda i: (0, i)),
        out_shape=jax.ShapeDtypeStruct((M, n_tok), jnp.float32),
        scratch_shapes=[pltpu.VMEM((D, M * B), jnp.bfloat16)],
    )(xf, v2)

    mesh = plsc.VectorSubcoreMesh(core_axis_name="c", subcore_axis_name="s")
    maskT = pl.kernel(
        _sc_top2,
        mesh=mesh,
        out_type=jax.ShapeDtypeStruct((M, n_tok), jnp.float32),
        scratch_types=[
            pltpu.VMEM((M, 128), jnp.float32),
            pltpu.VMEM((M, 128), jnp.float32),
        ],
    )(energyT)

    x_out, stats = pl.pallas_call(
        _stage_b,
        grid=(grid,),
        in_specs=[
            pl.BlockSpec((TN, D), lambda i: (i, 0)),
            pl.BlockSpec((D, M * B), lambda i: (0, 0)),
            pl.BlockSpec((D, M * B), lambda i: (0, 0)),
            pl.BlockSpec((1, M * B), lambda i: (0, 0)),
            pl.BlockSpec((M, TN), lambda i: (0, i)),
            pl.BlockSpec((M, TN), lambda i: (0, i)),
        ],
        out_specs=[
            pl.BlockSpec((TN, D), lambda i: (i, 0)),
            pl.BlockSpec(memory_space=pltpu.SMEM),
        ],
        out_shape=[
            jax.ShapeDtypeStruct((n_tok, D), jnp.float32),
            jax.ShapeDtypeStruct((8,), jnp.float32),
        ],
        scratch_shapes=[
            pltpu.VMEM((D, M * B), jnp.bfloat16),
            pltpu.VMEM((D, M * B), jnp.bfloat16),
            pltpu.VMEM((M, 1), jnp.float32),
            pltpu.VMEM((M, 1), jnp.float32),
            pltpu.SMEM((8,), jnp.float32),
        ],
    )(xf, v2, ud, ulast, energyT, maskT)

    x_out = x_out.reshape(x.shape)
    return (x_out, stats[0], stats[1], stats[2], stats[3], stats[4],
            stats[5], stats[6])
